# Initial kernel scaffold; baseline (speedup 1.0000x reference)
import jax
import jax.numpy as jnp
from jax.experimental import pallas as pl


def kernel(logits, top_ps, min_ps, top_ks, sampling_seed, positions):
    raise NotImplementedError("write your pallas kernel here")


# speedup vs baseline: 6.9890x; 6.9890x over previous
import jax
import jax.numpy as jnp
from jax.experimental import pallas as pl

_C = 1024
_R = 8


def _idx_cutoff_search(mask_src, tval, col, budget, V):
    R = tval.shape[0]
    is_tie = mask_src == tval
    lo = jnp.zeros((R, 1), jnp.int32)
    hi = jnp.full((R, 1), V, jnp.int32)

    def body(_, lh):
        lo, hi = lh
        mid = (lo + hi) // 2
        cnt = jnp.sum((is_tie & (col >= mid)).astype(jnp.int32), axis=1,
                      keepdims=True)
        ok = cnt <= budget
        return (jnp.where(ok, lo, mid + 1), jnp.where(ok, mid, hi))

    lo, hi = jax.lax.fori_loop(0, 17, body, (lo, hi))
    return hi


def _sampler_kernel(logits_ref, vals_ref, top_ps_ref, min_ps_ref,
                    top_ks_ref, seed_ref, pos_ref, ids_ref, out_ref):
    logits = logits_ref[...]
    vals = vals_ref[...]
    R, V = logits.shape
    C = vals.shape[1]

    m = vals[:, 0:1]
    p_full_un = jnp.exp(logits - m)
    d = jnp.sum(p_full_un, axis=1, keepdims=True)
    p_full = p_full_un / d
    p_sort = jnp.exp(vals - m) / d

    rank = jax.lax.broadcasted_iota(jnp.int32, (R, C), 1)
    tri = (jax.lax.broadcasted_iota(jnp.int32, (C, C), 0)
           <= jax.lax.broadcasted_iota(jnp.int32, (C, C), 1)).astype(jnp.float32)
    csum = jax.lax.dot_general(p_sort, tri, (((1,), (0,)), ((), ())),
                               precision=jax.lax.Precision.HIGHEST)

    top_p = top_ps_ref[...]
    min_p = min_ps_ref[...]
    top_k = top_ks_ref[...]
    keep = (rank < top_k) & ((csum - p_sort) <= top_p)
    thr = p_sort[:, 0:1] * min_p
    keep = keep & jnp.logical_not(p_sort < thr)
    p_masked = jnp.where(keep, p_sort, 0.0)
    n = jnp.sum(keep.astype(jnp.int32), axis=1, keepdims=True)

    seed = seed_ref[...]
    pos = pos_ref[...]
    step_seed = (seed * jnp.int32(19349663)) ^ (pos * jnp.int32(73856093))
    hashed = (step_seed * jnp.int32(-1)) ^ (rank * jnp.int32(479001599))
    u = (hashed & jnp.int32(0x00FFFFFF)).astype(jnp.float32) / float(2 ** 24)
    eps = 1e-10
    u = jnp.clip(u, eps, 1.0 - eps)
    gumbel = -jnp.log(-jnp.log(u))
    perturbed = jnp.log(p_masked + eps) + gumbel
    pmax = jnp.max(perturbed, axis=1, keepdims=True)
    s = jnp.min(jnp.where(perturbed == pmax, rank, C), axis=1, keepdims=True)

    col = jax.lax.broadcasted_iota(jnp.int32, (R, V), 1)

    tval = jnp.max(jnp.where(rank == s, p_sort, -1.0), axis=1, keepdims=True)
    f = jnp.sum((p_sort > tval).astype(jnp.int32), axis=1, keepdims=True)
    cut = _idx_cutoff_search(p_full, tval, col, s - f, V)
    ids_ref[...] = cut - 1

    t = jnp.max(jnp.where(rank == (n - 1), p_sort, -1.0), axis=1, keepdims=True)
    a = jnp.sum((p_sort > t).astype(jnp.int32), axis=1, keepdims=True)
    cut2 = _idx_cutoff_search(p_full, t, col, n - a, V)
    keep_orig = (p_full > t) | ((p_full == t) & (col >= cut2))
    out_ref[...] = jnp.where(keep_orig, p_full, 0.0)


@jax.jit
def kernel(logits, top_ps, min_ps, top_ks, sampling_seed, positions):
    B, V = logits.shape
    vals, _ = jax.lax.top_k(logits, _C)

    def rowvec(x, dtype):
        return x.astype(dtype).reshape(B, 1)

    ids, out = pl.pallas_call(
        _sampler_kernel,
        grid=(B // _R,),
        in_specs=[
            pl.BlockSpec((_R, V), lambda i: (i, 0)),
            pl.BlockSpec((_R, _C), lambda i: (i, 0)),
            pl.BlockSpec((_R, 1), lambda i: (i, 0)),
            pl.BlockSpec((_R, 1), lambda i: (i, 0)),
            pl.BlockSpec((_R, 1), lambda i: (i, 0)),
            pl.BlockSpec((_R, 1), lambda i: (i, 0)),
            pl.BlockSpec((_R, 1), lambda i: (i, 0)),
        ],
        out_specs=[
            pl.BlockSpec((_R, 1), lambda i: (i, 0)),
            pl.BlockSpec((_R, V), lambda i: (i, 0)),
        ],
        out_shape=[
            jax.ShapeDtypeStruct((B, 1), jnp.int32),
            jax.ShapeDtypeStruct((B, V), jnp.float32),
        ],
    )(logits, vals,
      rowvec(top_ps, jnp.float32), rowvec(min_ps, jnp.float32),
      rowvec(top_ks, jnp.int32), rowvec(sampling_seed, jnp.int32),
      rowvec(positions, jnp.int32))
    return ids.reshape(B), out
